# Initial kernel scaffold; baseline (speedup 1.0000x reference)
import jax
import jax.numpy as jnp
from jax.experimental import pallas as pl


def kernel(x, edge_index, edge_weight, Wr, br, Wu, bu, Wc, bc, Wf, bf, Wi, bi, Wg, bg, Wo, bo, alpha, Wro, bro):
    raise NotImplementedError("write your pallas kernel here")


# speedup vs baseline: 2.8035x; 2.8035x over previous
import functools

import jax
import jax.numpy as jnp
from jax import lax
from jax.experimental import pallas as pl
from jax.experimental.pallas import tpu as pltpu
from jax.experimental.pallas import tpu_sc as plsc

N = 10000
E = 160000
HID = 64
INF = 16
S = 6

NC = 2
NS = 16
LANES = 16

C = 128
EPT = E // NS
FULL_T = EPT // C
TAIL_T = EPT - FULL_T * C
EPW = E // (NC * NS)
FULL_W = EPW // C
TAIL_W = EPW - FULL_W * C
NP = 10240
NPT = NP // NS

_mesh = plsc.VectorSubcoreMesh(core_axis_name="c", subcore_axis_name="s")


def _scale_rows(rows, wrep, n_edges, fc):
    for e in range(n_edges):
        wb = wrep[e, pl.ds(0, LANES)]
        for j in range(fc // LANES):
            sl = (e, pl.ds(j * LANES, LANES))
            rows[sl] = rows[sl] * wb


def _make_propagate(fc):
    @functools.partial(
        pl.kernel,
        out_type=jax.ShapeDtypeStruct((NC, NP, fc), jnp.float32),
        mesh=_mesh,
        scratch_types=[
            pltpu.VMEM_SHARED((NP, fc), jnp.float32),
            pltpu.VMEM_SHARED((NP, fc), jnp.float32),
            pltpu.VMEM((C,), jnp.int32),
            pltpu.VMEM((C,), jnp.int32),
            pltpu.VMEM((C, LANES), jnp.float32),
            pltpu.VMEM((C, fc), jnp.float32),
            pltpu.VMEM((TAIL_T,), jnp.int32),
            pltpu.VMEM((TAIL_T,), jnp.int32),
            pltpu.VMEM((TAIL_T, LANES), jnp.float32),
            pltpu.VMEM((TAIL_T, fc), jnp.float32),
        ],
    )
    def prop(x_hbm, src_hbm, dst_hbm, w_hbm, zero_hbm, out_hbm,
             acc, stage, srcv, dstv, wv, rows, tsrc, tdst, tw, trows):
        c = lax.axis_index("c")
        s = lax.axis_index("s")
        stripe = pl.ds(s * NPT, NPT)
        pltpu.sync_copy(zero_hbm.at[stripe], acc.at[stripe])
        pltpu.sync_copy(x_hbm.at[c].at[stripe], stage.at[stripe])
        plsc.subcore_barrier()

        ebase = s * EPT

        def chunk(k, carry):
            base = ebase + k * C
            pltpu.sync_copy(src_hbm.at[pl.ds(base, C)], srcv)
            pltpu.sync_copy(dst_hbm.at[pl.ds(base, C)], dstv)
            pltpu.sync_copy(w_hbm.at[pl.ds(base, C)], wv)
            pltpu.sync_copy(stage.at[srcv], rows)
            _scale_rows(rows, wv, C, fc)
            pltpu.sync_copy(rows, acc.at[dstv], add=True)
            return carry

        lax.fori_loop(0, FULL_T, chunk, 0)

        tb = ebase + FULL_T * C
        pltpu.sync_copy(src_hbm.at[pl.ds(tb, TAIL_T)], tsrc)
        pltpu.sync_copy(dst_hbm.at[pl.ds(tb, TAIL_T)], tdst)
        pltpu.sync_copy(w_hbm.at[pl.ds(tb, TAIL_T)], tw)
        pltpu.sync_copy(stage.at[tsrc], trows)
        _scale_rows(trows, tw, TAIL_T, fc)
        pltpu.sync_copy(trows, acc.at[tdst], add=True)

        plsc.subcore_barrier()
        pltpu.sync_copy(acc.at[pl.ds(s * NPT, NPT)],
                        out_hbm.at[c].at[pl.ds(s * NPT, NPT)])

    return prop


@functools.partial(
    pl.kernel,
    out_type=jax.ShapeDtypeStruct((E, LANES), jnp.float32),
    mesh=_mesh,
    scratch_types=[
        pltpu.VMEM_SHARED((NP, LANES), jnp.float32),
        pltpu.VMEM((C,), jnp.int32),
        pltpu.VMEM((C, LANES), jnp.float32),
        pltpu.VMEM((C, LANES), jnp.float32),
        pltpu.VMEM((TAIL_T,), jnp.int32),
        pltpu.VMEM((TAIL_T, LANES), jnp.float32),
        pltpu.VMEM((TAIL_W,), jnp.int32),
        pltpu.VMEM((TAIL_W, LANES), jnp.float32),
        pltpu.VMEM((TAIL_W, LANES), jnp.float32),
    ],
)
def _norm_weights(ew_hbm, dst_hbm, zero_hbm, w_hbm,
                  deg, dstv, ewv, degv, tdst, tew, bdst, bew, bdeg):
    c = lax.axis_index("c")
    s = lax.axis_index("s")
    pltpu.sync_copy(zero_hbm.at[pl.ds(s * NPT, NPT)], deg.at[pl.ds(s * NPT, NPT)])
    plsc.subcore_barrier()

    ebase = s * EPT

    def chunk_a(k, carry):
        base = ebase + k * C
        pltpu.sync_copy(dst_hbm.at[pl.ds(base, C)], dstv)
        pltpu.sync_copy(ew_hbm.at[pl.ds(base, C)], ewv)
        pltpu.sync_copy(ewv, deg.at[dstv], add=True)
        return carry

    lax.fori_loop(0, FULL_T, chunk_a, 0)
    tb = ebase + FULL_T * C
    pltpu.sync_copy(dst_hbm.at[pl.ds(tb, TAIL_T)], tdst)
    pltpu.sync_copy(ew_hbm.at[pl.ds(tb, TAIL_T)], tew)
    pltpu.sync_copy(tew, deg.at[tdst], add=True)

    plsc.subcore_barrier()

    wbase = (s * NC + c) * EPW

    def chunk_b(k, carry):
        base = wbase + k * C
        pltpu.sync_copy(dst_hbm.at[pl.ds(base, C)], dstv)
        pltpu.sync_copy(ew_hbm.at[pl.ds(base, C)], ewv)
        pltpu.sync_copy(deg.at[dstv], degv)
        for e in range(C):
            sl = (e, pl.ds(0, LANES))
            ewv[sl] = ewv[sl] / jnp.maximum(degv[sl], 1e-6)
        pltpu.sync_copy(ewv, w_hbm.at[pl.ds(base, C)])
        return carry

    lax.fori_loop(0, FULL_W, chunk_b, 0)
    tb2 = wbase + FULL_W * C
    pltpu.sync_copy(dst_hbm.at[pl.ds(tb2, TAIL_W)], bdst)
    pltpu.sync_copy(ew_hbm.at[pl.ds(tb2, TAIL_W)], bew)
    pltpu.sync_copy(deg.at[bdst], bdeg)
    for e in range(TAIL_W):
        sl = (e, pl.ds(0, LANES))
        bew[sl] = bew[sl] / jnp.maximum(bdeg[sl], 1e-6)
    pltpu.sync_copy(bew, w_hbm.at[pl.ds(tb2, TAIL_W)])


BN = 1000
GRID = N // BN


def _wspec(shape):
    nd = len(shape)
    return pl.BlockSpec(shape, lambda i, _nd=nd: (0,) * _nd)


def _rspec(*shape):
    rest = len(shape) - 1
    return pl.BlockSpec(shape, lambda i, _r=rest: (i,) + (0,) * _r)


def _sspec(fc):
    return pl.BlockSpec((NC, BN, fc), lambda i: (0, i, 0))


def _tc_a_body(h_ref, Wf, bf, Wi1, bi, xs1_ref, z_ref):
    h = h_ref[...]
    xs1 = h @ Wf[...] + bf[...]
    z = xs1 @ Wi1[...] + bi[...]
    xs1_ref[...] = xs1
    z_ref[0] = z[:, :32]
    z_ref[1] = z[:, 32:]


def _tc_b_body(pz_ref, h_ref, Wg, bg, Wo, bo, alpha, Wro, bro,
               xs2_ref, rep_ref, xh_ref):
    pz = jnp.concatenate([pz_ref[0], pz_ref[1]], axis=-1)
    h = h_ref[...]
    conv = pz @ Wg[...] + bg[...]
    o = jnp.concatenate([conv, h], axis=-1) @ Wo[...] + bo[...]
    a = alpha[0]
    o = jnp.where(o > 0, o, a * o)
    rep = jnp.concatenate([o, h], axis=-1)
    xs2 = rep @ Wro[...] + bro[...]
    xs2_ref[...] = xs2
    rep_ref[...] = rep
    xh_ref[0] = jnp.concatenate([xs2, jnp.zeros((BN, INF), jnp.float32),
                                 h[:, :INF]], axis=-1)
    xh_ref[1] = h[:, INF:]


def _tc_c_body(xh_ref, axh_ref, aaxh_ref, h_ref,
               Wr0, Wr1, Wr2, br, Wu0, Wu1, Wu2, bu, u_ref, xc_ref):
    xh = jnp.concatenate([xh_ref[0], xh_ref[1]], axis=-1)
    axh = jnp.concatenate([axh_ref[0], axh_ref[1]], axis=-1)
    aaxh = jnp.concatenate([aaxh_ref[0], aaxh_ref[1]], axis=-1)
    h = h_ref[...]
    r = jax.nn.sigmoid(xh @ Wr0[...] + axh @ Wr1[...] + aaxh @ Wr2[...] + br[...])
    u_ref[...] = jax.nn.sigmoid(xh @ Wu0[...] + axh @ Wu1[...] + aaxh @ Wu2[...] + bu[...])
    rh = r * h
    xc_ref[0] = jnp.concatenate([xh_ref[0][:, :32], rh[:, :INF]], axis=-1)
    xc_ref[1] = rh[:, INF:]


def _tc_d_body(xc_ref, axc_ref, aaxc_ref, u_ref, h_ref,
               Wc0, Wc1, Wc2, bc, hn_ref):
    xc = jnp.concatenate([xc_ref[0], xc_ref[1]], axis=-1)
    axc = jnp.concatenate([axc_ref[0], axc_ref[1]], axis=-1)
    aaxc = jnp.concatenate([aaxc_ref[0], aaxc_ref[1]], axis=-1)
    u = u_ref[...]
    h = h_ref[...]
    cg = jnp.tanh(xc @ Wc0[...] + axc @ Wc1[...] + aaxc @ Wc2[...] + bc[...])
    hn_ref[...] = u * h + (1.0 - u) * cg


def _f32(*shape):
    return jax.ShapeDtypeStruct(shape, jnp.float32)


def kernel(x, edge_index, edge_weight, Wr, br, Wu, bu, Wc, bc, Wf, bf,
           Wi, bi, Wg, bg, Wo, bo, alpha, Wro, bro):
    src = edge_index[0].astype(jnp.int32)
    dst = edge_index[1].astype(jnp.int32)
    ew16 = jnp.broadcast_to(edge_weight[:, None], (E, LANES)).astype(jnp.float32)
    zeros16 = jnp.zeros((NP, LANES), jnp.float32)
    zeros32 = jnp.zeros((NP, 32), jnp.float32)
    zeros48 = jnp.zeros((NP, 48), jnp.float32)

    w16 = _norm_weights(ew16, dst, zeros16)

    prop32 = _make_propagate(32)
    prop48 = _make_propagate(48)

    gi = 2 * INF + HID
    Wr0, Wr1, Wr2 = Wr[:gi], Wr[gi:2 * gi], Wr[2 * gi:]
    Wu0, Wu1, Wu2 = Wu[:gi], Wu[gi:2 * gi], Wu[2 * gi:]
    Wc0, Wc1, Wc2 = Wc[:gi], Wc[gi:2 * gi], Wc[2 * gi:]
    Wi1 = Wi[:HID]

    tc_a = pl.pallas_call(
        _tc_a_body, grid=(GRID,),
        in_specs=[_rspec(BN, HID), _wspec((HID, HID)), _wspec((HID,)),
                  _wspec((HID, HID)), _wspec((HID,))],
        out_specs=[_rspec(BN, HID), _sspec(32)],
        out_shape=[_f32(N, HID), _f32(NC, NP, 32)],
    )
    tc_b = pl.pallas_call(
        _tc_b_body, grid=(GRID,),
        in_specs=[_sspec(32), _rspec(BN, HID), _wspec((HID, HID)), _wspec((HID,)),
                  _wspec((2 * HID, HID)), _wspec((HID,)), _wspec((1,)),
                  _wspec((2 * HID, INF)), _wspec((INF,))],
        out_specs=[_rspec(BN, INF), _rspec(BN, 2 * HID), _sspec(48)],
        out_shape=[_f32(N, INF), _f32(N, 2 * HID), _f32(NC, NP, 48)],
    )
    tc_c = pl.pallas_call(
        _tc_c_body, grid=(GRID,),
        in_specs=[_sspec(48), _sspec(48), _sspec(48), _rspec(BN, HID)]
                 + [_wspec((gi, HID)), _wspec((gi, HID)), _wspec((gi, HID)),
                    _wspec((HID,))] * 2,
        out_specs=[_rspec(BN, HID), _sspec(48)],
        out_shape=[_f32(N, HID), _f32(NC, NP, 48)],
    )
    tc_d = pl.pallas_call(
        _tc_d_body, grid=(GRID,),
        in_specs=[_sspec(48), _sspec(48), _sspec(48), _rspec(BN, HID),
                  _rspec(BN, HID), _wspec((gi, HID)), _wspec((gi, HID)),
                  _wspec((gi, HID)), _wspec((HID,))],
        out_specs=_rspec(BN, HID),
        out_shape=_f32(N, HID),
    )

    h = jnp.zeros((N, HID), jnp.float32)
    gens, preds, reprs, states = [], [], [], []
    for _ in range(S):
        xs1, z = tc_a(h, Wf, bf, Wi1, bi)
        pz = prop32(z, src, dst, w16, zeros32)
        xs2, rep, xh = tc_b(pz, h, Wg, bg, Wo, bo, alpha, Wro, bro)
        axh = prop48(xh, src, dst, w16, zeros48)
        aaxh = prop48(axh, src, dst, w16, zeros48)
        u, xc = tc_c(xh, axh, aaxh, h, Wr0, Wr1, Wr2, br, Wu0, Wu1, Wu2, bu)
        axc = prop48(xc, src, dst, w16, zeros48)
        aaxc = prop48(axc, src, dst, w16, zeros48)
        h = tc_d(xc, axc, aaxc, u, h, Wc0, Wc1, Wc2, bc)
        gens.append(xs2)
        preds.append(xs1)
        reprs.append(rep)
        states.append(h)

    generations = jnp.stack(gens, axis=0)[None]
    predictions = jnp.stack(preds, axis=0)[None]
    representations = jnp.stack(reprs, axis=0)[None]
    states_out = jnp.stack(states, axis=0)[None, None]
    return generations, predictions, representations, states_out
